# baseline (speedup 1.0000x reference)
import functools

import jax
import jax.numpy as jnp
from jax import lax
from jax.experimental import pallas as pl
from jax.experimental.pallas import tpu as pltpu
from jax.experimental.pallas import tpu_sc as plsc

_NC, _NS = 2, 16
_NW = _NC * _NS


def _make_sc_fill(num_tokens):
    per_w = num_tokens // _NW
    mesh = plsc.VectorSubcoreMesh(core_axis_name="c", subcore_axis_name="s")

    @functools.partial(
        pl.kernel,
        mesh=mesh,
        out_type=(
            jax.ShapeDtypeStruct((num_tokens,), jnp.int32),
            jax.ShapeDtypeStruct((num_tokens,), jnp.int32),
        ),
        scratch_types=[
            pltpu.VMEM((per_w,), jnp.int32),
            pltpu.VMEM((per_w,), jnp.int32),
        ],
    )
    def k(z_hbm, o_hbm, zv, ov):
        wid = lax.axis_index("s") * _NC + lax.axis_index("c")
        base = wid * per_w
        zv[...] = jnp.zeros_like(zv)
        ov[...] = jnp.ones_like(ov)
        pltpu.sync_copy(zv, z_hbm.at[pl.ds(base, per_w)])
        pltpu.sync_copy(ov, o_hbm.at[pl.ds(base, per_w)])

    return k


def kernel(hidden_states, routing_vectors):
    del routing_vectors
    num_tokens = hidden_states.shape[0]
    zeros, ones = _make_sc_fill(num_tokens)()
    return (zeros.reshape(num_tokens, 1), ones.reshape(num_tokens, 1))


# speedup vs baseline: 16.4815x; 16.4815x over previous
import jax
import jax.numpy as jnp
from jax.experimental import pallas as pl


def _gate_kernel(idx_ref, w_ref):
    idx_ref[...] = jnp.zeros_like(idx_ref)
    w_ref[...] = jnp.ones_like(w_ref)


def kernel(hidden_states, routing_vectors):
    del routing_vectors
    num_tokens = hidden_states.shape[0]
    out_shape = jax.ShapeDtypeStruct((num_tokens // 128, 128), jnp.int32)
    zeros, ones = pl.pallas_call(
        _gate_kernel,
        out_shape=(out_shape, out_shape),
    )()
    return (zeros.reshape(num_tokens, 1), ones.reshape(num_tokens, 1))
